# baseline (speedup 1.0000x reference)
import jax
import jax.numpy as jnp
from jax import lax
from jax.experimental import pallas as pl
from jax.experimental.pallas import tpu as pltpu
from jax.experimental.pallas import tpu_sc as plsc

NC = 2
NS = 16
NW = NC * NS

BT = 4096
SEQ = 200
D = 32
B = BT * SEQ
BPW = B // NW

_MESH = plsc.VectorSubcoreMesh(core_axis_name="c", subcore_axis_name="s")


def _worker_id():
    return lax.axis_index("s") * NC + lax.axis_index("c")


def _idx_transpose_body(idx_hbm, out_hbm, src_v, dst_v):
    w = _worker_id()
    base = w * BPW
    pltpu.sync_copy(idx_hbm.at[pl.ds(base, BPW)], src_v)
    iota200 = lax.iota(jnp.int32, 16) * 200

    def it(k, c):
        s = k >> 3
        jg = k & 7
        v = plsc.load_gather(src_v, [iota200 + (jg * 3200 + s)])
        dst_v[pl.ds(s * 128 + jg * 16, 16)] = v
        return c

    lax.fori_loop(0, SEQ * 8, it, 0)
    pltpu.sync_copy(dst_v, out_hbm.at[pl.ds(base, BPW)])


G_CHUNK = 1024
G_NCH = BPW // G_CHUNK
G_NBUF = 3


def _gather_body(idx_hbm, table_hbm, out_hbm, idx_v, r0, r1, r2,
                 g0, g1, g2, s0, s1, s2):
    rows = (r0, r1, r2)
    gsem = (g0, g1, g2)
    ssem = (s0, s1, s2)
    base = _worker_id() * BPW
    pltpu.sync_copy(idx_hbm.at[pl.ds(base, BPW)], idx_v)

    def start_gather(j, b):
        return pltpu.async_copy(
            table_hbm.at[idx_v.at[pl.ds(j * G_CHUNK, G_CHUNK)]],
            rows[b], gsem[b])

    def start_store(i, b):
        return pltpu.async_copy(
            rows[b], out_hbm.at[pl.ds(base + i * G_CHUNK, G_CHUNK)], ssem[b])

    pend_g = {0: start_gather(0, 0), 1: start_gather(1, 1)}
    pend_s = {}
    for i in range(G_NCH):
        b = i % G_NBUF
        pend_g[i].wait()
        pend_s[i] = start_store(i, b)
        j = i + 2
        if j < G_NCH:
            bj = j % G_NBUF
            if j >= G_NBUF:
                pend_s[j - G_NBUF].wait()
            pend_g[j] = start_gather(j, bj)
    for i in range(max(0, G_NCH - G_NBUF), G_NCH):
        pend_s[i].wait()


T_SC = 4
T_CR = T_SC * 128
T_NCH = SEQ // T_SC
RPITCH = D + 1


def _row_transpose_body(rows_hbm, out_hbm, rc0, rc1, rp, tb0, tb1,
                        l0, l1, o0, o1):
    rcb = (rc0, rc1)
    tbb = (tb0, tb1)
    lsem = (l0, l1)
    osem = (o0, o1)
    w = _worker_id()
    base = w * BPW
    iota = lax.iota(jnp.int32, 16)

    def start_load(g, b):
        return pltpu.async_copy(
            rows_hbm.at[pl.ds(base + g * T_CR, T_CR)], rcb[b], lsem[b])

    def wait_load(g, b):
        pltpu.make_async_copy(
            rows_hbm.at[pl.ds(base + g * T_CR, T_CR)], rcb[b],
            lsem[b]).wait()

    def start_store(g, b):
        return pltpu.async_copy(
            tbb[b], out_hbm.at[pl.ds(g * T_SC, T_SC), :, pl.ds(w, 1)],
            osem[b])

    def wait_store(g, b):
        pltpu.make_async_copy(
            tbb[b], out_hbm.at[pl.ds(g * T_SC, T_SC), :, pl.ds(w, 1)],
            osem[b]).wait()

    def pad_rows(b):
        rc = rcb[b]

        def pr(k, c):
            for u in range(4):
                r = k * 4 + u
                rp[r, pl.ds(0, 16)] = rc[r, pl.ds(0, 16)]
                rp[r, pl.ds(16, 16)] = rc[r, pl.ds(16, 16)]
            return c

        lax.fori_loop(0, T_CR // 4, pr, 0)

    def transpose(b):
        tb = tbb[b]

        def tr(k, c):
            s_ = k >> 3
            jg = k & 7
            row = iota + (s_ * 128 + jg * 16)
            for d in range(D):
                col = jnp.full((16,), d, jnp.int32)
                v = plsc.load_gather(rp, [row, col])
                tb[s_, d >> 3, 0, d & 7, pl.ds(jg * 16, 16)] = v
            return c

        lax.fori_loop(0, T_SC * 8, tr, 0)

    start_load(0, 0)
    start_load(1, 1)

    def chunk_pair(i, c):
        for h in range(2):
            g = i * 2 + h
            wait_load(g, h)
            pad_rows(h)

            @pl.when(g + 2 < T_NCH)
            def _():
                start_load(g + 2, h)

            @pl.when(g >= 2)
            def _():
                wait_store(g - 2, h)

            transpose(h)
            start_store(g, h)
        return c

    lax.fori_loop(0, T_NCH // 2, chunk_pair, 0)
    wait_store(T_NCH - 2, 0)
    wait_store(T_NCH - 1, 1)


@jax.jit
def kernel(inputs, embeddings):
    idx_flat = inputs.reshape(B).astype(jnp.int32)

    idx_sm = pl.kernel(
        _idx_transpose_body,
        out_type=jax.ShapeDtypeStruct((B,), jnp.int32),
        mesh=_MESH,
        scratch_types=[
            pltpu.VMEM((BPW,), jnp.int32),
            pltpu.VMEM((BPW,), jnp.int32),
        ],
        compiler_params=pltpu.CompilerParams(
            use_tc_tiling_on_sc=False, needs_layout_passes=False),
    )(idx_flat)

    rows = pl.kernel(
        _gather_body,
        out_type=jax.ShapeDtypeStruct((B, D), jnp.float32),
        mesh=_MESH,
        scratch_types=[
            pltpu.VMEM((BPW,), jnp.int32),
            pltpu.VMEM((G_CHUNK, D), jnp.float32),
            pltpu.VMEM((G_CHUNK, D), jnp.float32),
            pltpu.VMEM((G_CHUNK, D), jnp.float32),
            pltpu.SemaphoreType.DMA,
            pltpu.SemaphoreType.DMA,
            pltpu.SemaphoreType.DMA,
            pltpu.SemaphoreType.DMA,
            pltpu.SemaphoreType.DMA,
            pltpu.SemaphoreType.DMA,
        ],
        compiler_params=pltpu.CompilerParams(use_tc_tiling_on_sc=False),
    )(idx_sm, embeddings)

    out5 = pl.kernel(
        _row_transpose_body,
        out_type=jax.ShapeDtypeStruct((SEQ, D // 8, NW, 8, 128), jnp.float32),
        mesh=_MESH,
        scratch_types=[
            pltpu.VMEM((T_CR, D), jnp.float32),
            pltpu.VMEM((T_CR, D), jnp.float32),
            pltpu.VMEM((T_CR, RPITCH), jnp.float32),
            pltpu.VMEM((T_SC, D // 8, 1, 8, 128), jnp.float32),
            pltpu.VMEM((T_SC, D // 8, 1, 8, 128), jnp.float32),
            pltpu.SemaphoreType.DMA,
            pltpu.SemaphoreType.DMA,
            pltpu.SemaphoreType.DMA,
            pltpu.SemaphoreType.DMA,
        ],
        compiler_params=pltpu.CompilerParams(
            use_tc_tiling_on_sc=False, needs_layout_passes=False),
    )(rows)

    return out5.transpose(2, 4, 0, 1, 3).reshape(BT, SEQ, D)


# speedup vs baseline: 1.0535x; 1.0535x over previous
import jax
import jax.numpy as jnp
from jax import lax
from jax.experimental import pallas as pl
from jax.experimental.pallas import tpu as pltpu
from jax.experimental.pallas import tpu_sc as plsc

NC = 2
NS = 16
NW = NC * NS

BT = 4096
SEQ = 200
D = 32
B = BT * SEQ
BPW = B // NW

_MESH = plsc.VectorSubcoreMesh(core_axis_name="c", subcore_axis_name="s")


def _worker_id():
    return lax.axis_index("s") * NC + lax.axis_index("c")


def _idx_transpose_body(idx_hbm, out_hbm, src_v, dst_v):
    w = _worker_id()
    base = w * BPW
    pltpu.sync_copy(idx_hbm.at[pl.ds(base, BPW)], src_v)
    iota200 = lax.iota(jnp.int32, 16) * 200

    def it(k, c):
        s = k >> 3
        jg = k & 7
        v = plsc.load_gather(src_v, [iota200 + (jg * 3200 + s)])
        dst_v[pl.ds(s * 128 + jg * 16, 16)] = v
        return c

    lax.fori_loop(0, SEQ * 8, it, 0)
    pltpu.sync_copy(dst_v, out_hbm.at[pl.ds(base, BPW)])


G_CHUNK = 1024
G_NCH = BPW // G_CHUNK
G_NBUF = 3


def _gather_body(idx_hbm, table_hbm, out_hbm, idx_v, r0, r1, r2,
                 g0, g1, g2, s0, s1, s2):
    rows = (r0, r1, r2)
    gsem = (g0, g1, g2)
    ssem = (s0, s1, s2)
    base = _worker_id() * BPW
    pltpu.sync_copy(idx_hbm.at[pl.ds(base, BPW)], idx_v)

    def start_gather(j, b):
        return pltpu.async_copy(
            table_hbm.at[idx_v.at[pl.ds(j * G_CHUNK, G_CHUNK)]],
            rows[b], gsem[b])

    def start_store(i, b):
        return pltpu.async_copy(
            rows[b], out_hbm.at[pl.ds(base + i * G_CHUNK, G_CHUNK)], ssem[b])

    pend_g = {0: start_gather(0, 0), 1: start_gather(1, 1)}
    pend_s = {}
    for i in range(G_NCH):
        b = i % G_NBUF
        pend_g[i].wait()
        pend_s[i] = start_store(i, b)
        j = i + 2
        if j < G_NCH:
            bj = j % G_NBUF
            if j >= G_NBUF:
                pend_s[j - G_NBUF].wait()
            pend_g[j] = start_gather(j, bj)
    for i in range(max(0, G_NCH - G_NBUF), G_NCH):
        pend_s[i].wait()


T_SC = 4
T_CR = T_SC * 128
T_NCH = SEQ // T_SC
RPITCH = D + 1


def _row_transpose_body(rows_hbm, out_hbm, rp0, rp1, tb0, tb1,
                        l0, l1, o0, o1):
    rpb = (rp0, rp1)
    tbb = (tb0, tb1)
    lsem = (l0, l1)
    osem = (o0, o1)
    w = _worker_id()
    base = w * BPW
    iota = lax.iota(jnp.int32, 16)

    def start_load(g, b):
        return pltpu.async_copy(
            rows_hbm.at[pl.ds(base + g * T_CR, T_CR)],
            rpb[b].at[:, pl.ds(0, D)], lsem[b])

    def wait_load(g, b):
        pltpu.make_async_copy(
            rows_hbm.at[pl.ds(base + g * T_CR, T_CR)],
            rpb[b].at[:, pl.ds(0, D)], lsem[b]).wait()

    def start_store(g, b):
        return pltpu.async_copy(
            tbb[b], out_hbm.at[pl.ds(g * T_SC, T_SC), :, pl.ds(w, 1)],
            osem[b])

    def wait_store(g, b):
        pltpu.make_async_copy(
            tbb[b], out_hbm.at[pl.ds(g * T_SC, T_SC), :, pl.ds(w, 1)],
            osem[b]).wait()

    def transpose(b):
        rp = rpb[b]
        tb = tbb[b]

        def tr(k, c):
            s_ = k >> 3
            jg = k & 7
            row = iota + (s_ * 128 + jg * 16)
            for d in range(D):
                col = jnp.full((16,), d, jnp.int32)
                v = plsc.load_gather(rp, [row, col])
                tb[s_, d >> 3, 0, d & 7, pl.ds(jg * 16, 16)] = v
            return c

        lax.fori_loop(0, T_SC * 8, tr, 0)

    start_load(0, 0)
    start_load(1, 1)

    def chunk_pair(i, c):
        for h in range(2):
            g = i * 2 + h
            wait_load(g, h)

            @pl.when(g >= 2)
            def _():
                wait_store(g - 2, h)

            transpose(h)

            @pl.when(g + 2 < T_NCH)
            def _():
                start_load(g + 2, h)

            start_store(g, h)
        return c

    lax.fori_loop(0, T_NCH // 2, chunk_pair, 0)
    wait_store(T_NCH - 2, 0)
    wait_store(T_NCH - 1, 1)


@jax.jit
def kernel(inputs, embeddings):
    idx_flat = inputs.reshape(B).astype(jnp.int32)

    idx_sm = pl.kernel(
        _idx_transpose_body,
        out_type=jax.ShapeDtypeStruct((B,), jnp.int32),
        mesh=_MESH,
        scratch_types=[
            pltpu.VMEM((BPW,), jnp.int32),
            pltpu.VMEM((BPW,), jnp.int32),
        ],
        compiler_params=pltpu.CompilerParams(
            use_tc_tiling_on_sc=False, needs_layout_passes=False),
    )(idx_flat)

    rows = pl.kernel(
        _gather_body,
        out_type=jax.ShapeDtypeStruct((B, D), jnp.float32),
        mesh=_MESH,
        scratch_types=[
            pltpu.VMEM((BPW,), jnp.int32),
            pltpu.VMEM((G_CHUNK, D), jnp.float32),
            pltpu.VMEM((G_CHUNK, D), jnp.float32),
            pltpu.VMEM((G_CHUNK, D), jnp.float32),
            pltpu.SemaphoreType.DMA,
            pltpu.SemaphoreType.DMA,
            pltpu.SemaphoreType.DMA,
            pltpu.SemaphoreType.DMA,
            pltpu.SemaphoreType.DMA,
            pltpu.SemaphoreType.DMA,
        ],
        compiler_params=pltpu.CompilerParams(use_tc_tiling_on_sc=False),
    )(idx_sm, embeddings)

    out5 = pl.kernel(
        _row_transpose_body,
        out_type=jax.ShapeDtypeStruct((SEQ, D // 8, NW, 8, 128), jnp.float32),
        mesh=_MESH,
        scratch_types=[
            pltpu.VMEM((T_CR, RPITCH), jnp.float32),
            pltpu.VMEM((T_CR, RPITCH), jnp.float32),
            pltpu.VMEM((T_SC, D // 8, 1, 8, 128), jnp.float32),
            pltpu.VMEM((T_SC, D // 8, 1, 8, 128), jnp.float32),
            pltpu.SemaphoreType.DMA,
            pltpu.SemaphoreType.DMA,
            pltpu.SemaphoreType.DMA,
            pltpu.SemaphoreType.DMA,
        ],
        compiler_params=pltpu.CompilerParams(
            use_tc_tiling_on_sc=False, needs_layout_passes=False,
            disable_bounds_checks=True),
    )(rows)

    return out5.transpose(2, 4, 0, 1, 3).reshape(BT, SEQ, D)


# speedup vs baseline: 1.0551x; 1.0015x over previous
import jax
import jax.numpy as jnp
from jax import lax
from jax.experimental import pallas as pl
from jax.experimental.pallas import tpu as pltpu
from jax.experimental.pallas import tpu_sc as plsc

NC = 2
NS = 16
NW = NC * NS

BT = 4096
SEQ = 200
D = 32
B = BT * SEQ
BPW = B // NW

_MESH = plsc.VectorSubcoreMesh(core_axis_name="c", subcore_axis_name="s")


def _worker_id():
    return lax.axis_index("s") * NC + lax.axis_index("c")


G_SROWS = 8
G_CHUNK = G_SROWS * 128
G_NCH = SEQ // G_SROWS
G_NBUF = 3


def _gather_body(inT_hbm, table_hbm, out_hbm, i0, i1, i2, r0, r1, r2,
                 n0, n1, n2, g0, g1, g2, s0, s1, s2):
    idxb = (i0, i1, i2)
    rows = (r0, r1, r2)
    isem = (n0, n1, n2)
    gsem = (g0, g1, g2)
    ssem = (s0, s1, s2)
    w = _worker_id()
    col = w * 128
    base = w * BPW

    def start_idx(j, b):
        for k in range(G_SROWS):
            pltpu.async_copy(
                inT_hbm.at[j * G_SROWS + k, pl.ds(col, 128)],
                idxb[b].at[pl.ds(k * 128, 128)], isem[b])

    def wait_idx(j, b):
        for k in range(G_SROWS):
            pltpu.make_async_copy(
                inT_hbm.at[j * G_SROWS + k, pl.ds(col, 128)],
                idxb[b].at[pl.ds(k * 128, 128)], isem[b]).wait()

    def start_gather(j, b):
        return pltpu.async_copy(table_hbm.at[idxb[b]], rows[b], gsem[b])

    def start_store(i, b):
        return pltpu.async_copy(
            rows[b], out_hbm.at[pl.ds(base + i * G_CHUNK, G_CHUNK)], ssem[b])

    start_idx(0, 0)
    start_idx(1, 1)
    wait_idx(0, 0)
    pend_g = {0: start_gather(0, 0)}
    start_idx(2, 2)
    wait_idx(1, 1)
    pend_g[1] = start_gather(1, 1)
    pend_s = {}
    for i in range(G_NCH):
        b = i % G_NBUF
        pend_g[i].wait()
        pend_s[i] = start_store(i, b)
        j = i + 2
        if j < G_NCH:
            bj = j % G_NBUF
            if j >= G_NBUF:
                pend_s[j - G_NBUF].wait()
            wait_idx(j, bj)
            pend_g[j] = start_gather(j, bj)
            if j + 1 < G_NCH:
                start_idx(j + 1, (j + 1) % G_NBUF)
    for i in range(max(0, G_NCH - G_NBUF), G_NCH):
        pend_s[i].wait()


T_SC = 4
T_CR = T_SC * 128
T_NCH = SEQ // T_SC
RPITCH = D + 1


def _row_transpose_body(rows_hbm, out_hbm, rp0, rp1, tb0, tb1,
                        l0, l1, o0, o1):
    rpb = (rp0, rp1)
    tbb = (tb0, tb1)
    lsem = (l0, l1)
    osem = (o0, o1)
    w = _worker_id()
    base = w * BPW
    iota = lax.iota(jnp.int32, 16)

    def start_load(g, b):
        return pltpu.async_copy(
            rows_hbm.at[pl.ds(base + g * T_CR, T_CR)],
            rpb[b].at[:, pl.ds(0, D)], lsem[b])

    def wait_load(g, b):
        pltpu.make_async_copy(
            rows_hbm.at[pl.ds(base + g * T_CR, T_CR)],
            rpb[b].at[:, pl.ds(0, D)], lsem[b]).wait()

    def start_store(g, b):
        return pltpu.async_copy(
            tbb[b], out_hbm.at[pl.ds(g * T_SC, T_SC), :, pl.ds(w, 1)],
            osem[b])

    def wait_store(g, b):
        pltpu.make_async_copy(
            tbb[b], out_hbm.at[pl.ds(g * T_SC, T_SC), :, pl.ds(w, 1)],
            osem[b]).wait()

    def transpose(b):
        rp = rpb[b]
        tb = tbb[b]

        def tr(k, c):
            s_ = k >> 3
            jg = k & 7
            row = iota + (s_ * 128 + jg * 16)
            for d in range(D):
                col = jnp.full((16,), d, jnp.int32)
                v = plsc.load_gather(rp, [row, col])
                tb[s_, d >> 3, 0, d & 7, pl.ds(jg * 16, 16)] = v
            return c

        lax.fori_loop(0, T_SC * 8, tr, 0)

    start_load(0, 0)
    start_load(1, 1)

    def chunk_pair(i, c):
        for h in range(2):
            g = i * 2 + h
            wait_load(g, h)

            @pl.when(g >= 2)
            def _():
                wait_store(g - 2, h)

            transpose(h)

            @pl.when(g + 2 < T_NCH)
            def _():
                start_load(g + 2, h)

            start_store(g, h)
        return c

    lax.fori_loop(0, T_NCH // 2, chunk_pair, 0)
    wait_store(T_NCH - 2, 0)
    wait_store(T_NCH - 1, 1)


@jax.jit
def kernel(inputs, embeddings):
    in_t = inputs.T.astype(jnp.int32)

    rows = pl.kernel(
        _gather_body,
        out_type=jax.ShapeDtypeStruct((B, D), jnp.float32),
        mesh=_MESH,
        scratch_types=[
            pltpu.VMEM((G_CHUNK,), jnp.int32),
            pltpu.VMEM((G_CHUNK,), jnp.int32),
            pltpu.VMEM((G_CHUNK,), jnp.int32),
            pltpu.VMEM((G_CHUNK, D), jnp.float32),
            pltpu.VMEM((G_CHUNK, D), jnp.float32),
            pltpu.VMEM((G_CHUNK, D), jnp.float32),
            pltpu.SemaphoreType.DMA,
            pltpu.SemaphoreType.DMA,
            pltpu.SemaphoreType.DMA,
            pltpu.SemaphoreType.DMA,
            pltpu.SemaphoreType.DMA,
            pltpu.SemaphoreType.DMA,
            pltpu.SemaphoreType.DMA,
            pltpu.SemaphoreType.DMA,
            pltpu.SemaphoreType.DMA,
        ],
        compiler_params=pltpu.CompilerParams(use_tc_tiling_on_sc=False),
    )(in_t, embeddings)

    out5 = pl.kernel(
        _row_transpose_body,
        out_type=jax.ShapeDtypeStruct((SEQ, D // 8, NW, 8, 128), jnp.float32),
        mesh=_MESH,
        scratch_types=[
            pltpu.VMEM((T_CR, RPITCH), jnp.float32),
            pltpu.VMEM((T_CR, RPITCH), jnp.float32),
            pltpu.VMEM((T_SC, D // 8, 1, 8, 128), jnp.float32),
            pltpu.VMEM((T_SC, D // 8, 1, 8, 128), jnp.float32),
            pltpu.SemaphoreType.DMA,
            pltpu.SemaphoreType.DMA,
            pltpu.SemaphoreType.DMA,
            pltpu.SemaphoreType.DMA,
        ],
        compiler_params=pltpu.CompilerParams(
            use_tc_tiling_on_sc=False, needs_layout_passes=False,
            disable_bounds_checks=True),
    )(rows)

    return out5.transpose(2, 4, 0, 1, 3).reshape(BT, SEQ, D)
